# baseline (speedup 1.0000x reference)
import functools

import jax
import jax.numpy as jnp
from jax import lax
from jax.experimental import pallas as pl
from jax.experimental.pallas import tpu as pltpu
from jax.experimental.pallas import tpu_sc as plsc

N = 10000
E = 320000
D = 128
H = 128
C = 64

NC = 2
NS = 16
NPAD = 10240
RPT = NPAD // NS

ECH = 128
AECH = 64
ANCH = 158
EP = NC * NS * ANCH * AECH
NPADROWS = NPAD - N

DNCH = EP // NS // ECH

_mesh = plsc.VectorSubcoreMesh(core_axis_name="c", subcore_axis_name="s",
                               num_cores=NC)


@functools.partial(
    pl.kernel,
    out_type=jax.ShapeDtypeStruct((2, NPAD), jnp.float32),
    mesh=_mesh,
    scratch_types=[
        pltpu.VMEM((DNCH, ECH), jnp.int32),
        pltpu.VMEM((ECH,), jnp.float32),
        pltpu.VMEM((RPT,), jnp.float32),
        pltpu.VMEM_SHARED((NPAD,), jnp.float32),
    ],
)
def _deg_kernel(edges_hbm, zeros_hbm, out_hbm, idx_v, ones_v, row_v, acc_sh):
    c = lax.axis_index("c")
    s = lax.axis_index("s")
    pltpu.sync_copy(zeros_hbm.at[pl.ds(s * RPT, RPT)],
                    acc_sh.at[pl.ds(s * RPT, RPT)])
    pltpu.sync_copy(edges_hbm.at[c, s], idx_v)
    ones16 = jnp.ones((16,), jnp.float32)

    def fill_ones(i, _):
        ones_v[pl.ds(i * 16, 16)] = ones16
        return 0

    lax.fori_loop(0, ECH // 16, fill_ones, 0)
    plsc.subcore_barrier()

    def body(j, _):
        pltpu.sync_copy(ones_v, acc_sh.at[idx_v.at[j]], add=True)
        return 0

    lax.fori_loop(0, DNCH, body, 0)
    plsc.subcore_barrier()
    pltpu.sync_copy(acc_sh.at[pl.ds(s * RPT, RPT)], row_v)
    pltpu.sync_copy(row_v, out_hbm.at[c, pl.ds(s * RPT, RPT)])


@functools.partial(
    pl.kernel,
    out_type=jax.ShapeDtypeStruct((NC, NPAD, D), jnp.float32),
    mesh=_mesh,
    scratch_types=[
        pltpu.VMEM((8, AECH), jnp.int32),
        pltpu.VMEM((8, AECH), jnp.int32),
        pltpu.VMEM((4, AECH, D), jnp.float32),
        pltpu.SemaphoreType.DMA((8,)),
        pltpu.SemaphoreType.DMA((8,)),
        pltpu.VMEM_SHARED((NPAD, D), jnp.float32),
    ],
)
def _agg_kernel(x_hbm, edges_hbm, zrows_hbm, out_hbm, src_v, dst_v, buf_v,
                gsem, isem, acc_sh):
    c = lax.axis_index("c")
    s = lax.axis_index("s")
    pltpu.sync_copy(zrows_hbm, acc_sh.at[pl.ds(s * RPT, RPT)])
    plsc.subcore_barrier()

    def fire_idx(q):
        qq = q % 8
        pltpu.async_copy(edges_hbm.at[0, c, s, q], src_v.at[qq], isem.at[qq])
        pltpu.async_copy(edges_hbm.at[1, c, s, q], dst_v.at[qq], isem.at[qq])

    def wait_idx(q):
        qq = lax.rem(q, 8)
        pltpu.make_async_copy(edges_hbm.at[0, c, s, q], src_v.at[qq],
                              isem.at[qq]).wait()
        pltpu.make_async_copy(edges_hbm.at[1, c, s, q], dst_v.at[qq],
                              isem.at[qq]).wait()

    HB = AECH // 2

    def fire_gather(q):
        qq = lax.rem(q, 8)
        qp = lax.rem(q, 4)
        for h in (0, 1):
            pltpu.async_copy(x_hbm.at[src_v.at[qq, pl.ds(h * HB, HB)]],
                             buf_v.at[qp, pl.ds(h * HB, HB)],
                             gsem.at[2 * qp + h])

    def wait_gather(q):
        qq = lax.rem(q, 8)
        qp = lax.rem(q, 4)
        for h in (0, 1):
            pltpu.make_async_copy(x_hbm.at[src_v.at[qq, pl.ds(h * HB, HB)]],
                                  buf_v.at[qp, pl.ds(h * HB, HB)],
                                  gsem.at[2 * qp + h]).wait()

    pltpu.sync_copy(edges_hbm.at[0, c, s, 0], src_v.at[0])
    pltpu.sync_copy(edges_hbm.at[1, c, s, 0], dst_v.at[0])
    for q in (1, 2, 3, 4):
        fire_idx(q)
    wait_idx(1)
    wait_idx(2)
    fire_gather(0)
    fire_gather(1)
    fire_gather(2)

    def body(j, _):
        p = lax.rem(j, 4)

        @pl.when(j + 3 < ANCH)
        def _():
            wait_idx(j + 3)
            fire_gather(j + 3)

        @pl.when(j + 5 < ANCH)
        def _():
            fire_idx(j + 5)

        wait_gather(j)
        pltpu.sync_copy(buf_v.at[p], acc_sh.at[dst_v.at[lax.rem(j, 8)]],
                        add=True)
        return 0

    lax.fori_loop(0, ANCH, body, 0)
    plsc.subcore_barrier()
    pltpu.sync_copy(acc_sh.at[pl.ds(s * RPT, RPT)],
                    out_hbm.at[c, pl.ds(s * RPT, RPT)])


def _scale_body(x_ref, co_ref, out_ref):
    so = lax.rsqrt(jnp.maximum(co_ref[...], 1.0))
    out_ref[:N, :] = x_ref[...] * so
    out_ref[N:, :] = jnp.zeros((NPAD - N, D), jnp.float32)


def _layer_body(m_ref, ci_ref, co_ref, w_ref, b_ref, g_ref, be_ref, out_ref):
    m = m_ref[0, :N, :] + m_ref[1, :N, :]
    m = m * lax.rsqrt(jnp.maximum(ci_ref[...], 1.0))
    z = jnp.dot(m, w_ref[...], preferred_element_type=jnp.float32) + b_ref[...]
    mu = jnp.mean(z, axis=0, keepdims=True)
    zc = z - mu
    var = jnp.mean(zc * zc, axis=0, keepdims=True)
    h = zc * lax.rsqrt(var + 1e-5) * g_ref[...] + be_ref[...]
    h = jnp.maximum(h, 0.0)
    out_ref[:N, :] = h * lax.rsqrt(jnp.maximum(co_ref[...], 1.0))
    out_ref[N:, :] = jnp.zeros((NPAD - N, H), jnp.float32)


def _final_body(m_ref, ci_ref, w_ref, b_ref, g_ref, be_ref, wc_ref, bc_ref,
                out_ref):
    m = m_ref[0, :N, :] + m_ref[1, :N, :]
    m = m * lax.rsqrt(jnp.maximum(ci_ref[...], 1.0))
    z = jnp.dot(m, w_ref[...], preferred_element_type=jnp.float32) + b_ref[...]
    mu = jnp.mean(z, axis=0, keepdims=True)
    zc = z - mu
    var = jnp.mean(zc * zc, axis=0, keepdims=True)
    h = zc * lax.rsqrt(var + 1e-5) * g_ref[...] + be_ref[...]
    h = jnp.maximum(h, 0.0)
    out_ref[...] = (jnp.dot(h, wc_ref[...], preferred_element_type=jnp.float32)
                    + bc_ref[...])


_scale = pl.pallas_call(
    _scale_body, out_shape=jax.ShapeDtypeStruct((NPAD, D), jnp.float32))
_layer = pl.pallas_call(
    _layer_body, out_shape=jax.ShapeDtypeStruct((NPAD, H), jnp.float32))
_final = pl.pallas_call(
    _final_body, out_shape=jax.ShapeDtypeStruct((N, C), jnp.float32))


@jax.jit
def kernel(x, edge_index, W1, b1, g1, be1, W2, b2, g2, be2, Wc, bc):
    pad = (N + (jnp.arange(EP - E, dtype=jnp.int32) % NPADROWS))[None, :]
    e_pad = jnp.concatenate([edge_index, jnp.broadcast_to(pad, (2, EP - E))],
                            axis=1)
    e_deg = e_pad.reshape(2, NS, DNCH, ECH)
    e_agg = e_pad.reshape(2, NC, NS, ANCH, AECH)
    zeros1 = jnp.zeros((NPAD,), jnp.float32)
    zrows = jnp.zeros((RPT, D), jnp.float32)

    cnt = _deg_kernel(e_deg, zeros1)
    co = cnt[0, :N].reshape(N, 1)
    ci = cnt[1, :N].reshape(N, 1)

    xs = _scale(x, co)
    m1 = _agg_kernel(xs, e_agg, zrows)
    h1 = _layer(m1, ci, co, W1, b1.reshape(1, H), g1.reshape(1, H),
                be1.reshape(1, H))
    m2 = _agg_kernel(h1, e_agg, zrows)
    out = _final(m2, ci, W2, b2.reshape(1, H), g2.reshape(1, H),
                 be2.reshape(1, H), Wc, bc.reshape(1, C))
    return out


# speedup vs baseline: 1.0358x; 1.0358x over previous
import functools

import jax
import jax.numpy as jnp
from jax import lax
from jax.experimental import pallas as pl
from jax.experimental.pallas import tpu as pltpu
from jax.experimental.pallas import tpu_sc as plsc

N = 10000
E = 320000
D = 128
H = 128
C = 64

NC = 2
NS = 16
NPAD = 10240
RPT = NPAD // NS

ECH = 128
AECH = 64
ANCH = 158
EP = NC * NS * ANCH * AECH
NPADROWS = NPAD - N

DNCH = EP // NS // ECH

_mesh = plsc.VectorSubcoreMesh(core_axis_name="c", subcore_axis_name="s",
                               num_cores=NC)


@functools.partial(
    pl.kernel,
    out_type=jax.ShapeDtypeStruct((2, NPAD), jnp.float32),
    mesh=_mesh,
    scratch_types=[
        pltpu.VMEM((DNCH, ECH), jnp.int32),
        pltpu.VMEM((ECH,), jnp.float32),
        pltpu.SemaphoreType.DMA((4,)),
        pltpu.VMEM((RPT,), jnp.float32),
        pltpu.VMEM_SHARED((NPAD,), jnp.float32),
    ],
)
def _deg_kernel(edges_hbm, zeros_hbm, out_hbm, idx_v, ones_v, ssem, row_v,
                acc_sh):
    c = lax.axis_index("c")
    s = lax.axis_index("s")
    pltpu.sync_copy(zeros_hbm.at[pl.ds(s * RPT, RPT)],
                    acc_sh.at[pl.ds(s * RPT, RPT)])
    pltpu.sync_copy(edges_hbm.at[c, s], idx_v)
    ones16 = jnp.ones((16,), jnp.float32)

    def fill_ones(i, _):
        ones_v[pl.ds(i * 16, 16)] = ones16
        return 0

    lax.fori_loop(0, ECH // 16, fill_ones, 0)
    plsc.subcore_barrier()

    def fire_sc(q):
        qq = lax.rem(q, 4)
        pltpu.async_copy(ones_v, acc_sh.at[idx_v.at[q]], ssem.at[qq])

    def wait_sc(q):
        qq = lax.rem(q, 4)
        pltpu.make_async_copy(ones_v, acc_sh.at[idx_v.at[q]],
                              ssem.at[qq]).wait()

    fire_sc(0)
    fire_sc(1)
    fire_sc(2)

    def body(j, _):
        @pl.when(j + 3 < DNCH)
        def _():
            fire_sc(j + 3)

        wait_sc(j)
        return 0

    lax.fori_loop(0, DNCH, body, 0)
    plsc.subcore_barrier()
    pltpu.sync_copy(acc_sh.at[pl.ds(s * RPT, RPT)], row_v)
    pltpu.sync_copy(row_v, out_hbm.at[c, pl.ds(s * RPT, RPT)])


@functools.partial(
    pl.kernel,
    out_type=jax.ShapeDtypeStruct((NC, NPAD, D), jnp.float32),
    mesh=_mesh,
    scratch_types=[
        pltpu.VMEM((8, AECH), jnp.int32),
        pltpu.VMEM((8, AECH), jnp.int32),
        pltpu.VMEM((4, AECH, D), jnp.float32),
        pltpu.SemaphoreType.DMA((8,)),
        pltpu.SemaphoreType.DMA((8,)),
        pltpu.VMEM_SHARED((NPAD, D), jnp.float32),
    ],
)
def _agg_kernel(x_hbm, edges_hbm, zrows_hbm, out_hbm, src_v, dst_v, buf_v,
                gsem, isem, acc_sh):
    c = lax.axis_index("c")
    s = lax.axis_index("s")
    pltpu.sync_copy(zrows_hbm, acc_sh.at[pl.ds(s * RPT, RPT)])
    plsc.subcore_barrier()

    def fire_idx(q):
        qq = q % 8
        pltpu.async_copy(edges_hbm.at[0, c, s, q], src_v.at[qq], isem.at[qq])
        pltpu.async_copy(edges_hbm.at[1, c, s, q], dst_v.at[qq], isem.at[qq])

    def wait_idx(q):
        qq = lax.rem(q, 8)
        pltpu.make_async_copy(edges_hbm.at[0, c, s, q], src_v.at[qq],
                              isem.at[qq]).wait()
        pltpu.make_async_copy(edges_hbm.at[1, c, s, q], dst_v.at[qq],
                              isem.at[qq]).wait()

    HB = AECH // 2

    def fire_gather(q):
        qq = lax.rem(q, 8)
        qp = lax.rem(q, 4)
        for h in (0, 1):
            pltpu.async_copy(x_hbm.at[src_v.at[qq, pl.ds(h * HB, HB)]],
                             buf_v.at[qp, pl.ds(h * HB, HB)],
                             gsem.at[2 * qp + h])

    def wait_gather(q):
        qq = lax.rem(q, 8)
        qp = lax.rem(q, 4)
        for h in (0, 1):
            pltpu.make_async_copy(x_hbm.at[src_v.at[qq, pl.ds(h * HB, HB)]],
                                  buf_v.at[qp, pl.ds(h * HB, HB)],
                                  gsem.at[2 * qp + h]).wait()

    pltpu.sync_copy(edges_hbm.at[0, c, s, 0], src_v.at[0])
    pltpu.sync_copy(edges_hbm.at[1, c, s, 0], dst_v.at[0])
    for q in (1, 2, 3, 4):
        fire_idx(q)
    wait_idx(1)
    wait_idx(2)
    fire_gather(0)
    fire_gather(1)
    fire_gather(2)

    def body(j, _):
        p = lax.rem(j, 4)

        @pl.when(j + 3 < ANCH)
        def _():
            wait_idx(j + 3)
            fire_gather(j + 3)

        @pl.when(j + 5 < ANCH)
        def _():
            fire_idx(j + 5)

        wait_gather(j)
        pltpu.sync_copy(buf_v.at[p], acc_sh.at[dst_v.at[lax.rem(j, 8)]],
                        add=True)
        return 0

    lax.fori_loop(0, ANCH, body, 0)
    plsc.subcore_barrier()
    pltpu.sync_copy(acc_sh.at[pl.ds(s * RPT, RPT)],
                    out_hbm.at[c, pl.ds(s * RPT, RPT)])


def _scale_body(x_ref, co_ref, out_ref):
    so = lax.rsqrt(jnp.maximum(co_ref[...], 1.0))
    out_ref[:N, :] = x_ref[...] * so
    out_ref[N:, :] = jnp.zeros((NPAD - N, D), jnp.float32)


def _layer_body(m_ref, ci_ref, co_ref, w_ref, b_ref, g_ref, be_ref, out_ref):
    m = m_ref[0, :N, :] + m_ref[1, :N, :]
    m = m * lax.rsqrt(jnp.maximum(ci_ref[...], 1.0))
    z = jnp.dot(m, w_ref[...], preferred_element_type=jnp.float32) + b_ref[...]
    mu = jnp.mean(z, axis=0, keepdims=True)
    zc = z - mu
    var = jnp.mean(zc * zc, axis=0, keepdims=True)
    h = zc * lax.rsqrt(var + 1e-5) * g_ref[...] + be_ref[...]
    h = jnp.maximum(h, 0.0)
    out_ref[:N, :] = h * lax.rsqrt(jnp.maximum(co_ref[...], 1.0))
    out_ref[N:, :] = jnp.zeros((NPAD - N, H), jnp.float32)


def _final_body(m_ref, ci_ref, w_ref, b_ref, g_ref, be_ref, wc_ref, bc_ref,
                out_ref):
    m = m_ref[0, :N, :] + m_ref[1, :N, :]
    m = m * lax.rsqrt(jnp.maximum(ci_ref[...], 1.0))
    z = jnp.dot(m, w_ref[...], preferred_element_type=jnp.float32) + b_ref[...]
    mu = jnp.mean(z, axis=0, keepdims=True)
    zc = z - mu
    var = jnp.mean(zc * zc, axis=0, keepdims=True)
    h = zc * lax.rsqrt(var + 1e-5) * g_ref[...] + be_ref[...]
    h = jnp.maximum(h, 0.0)
    out_ref[...] = (jnp.dot(h, wc_ref[...], preferred_element_type=jnp.float32)
                    + bc_ref[...])


_scale = pl.pallas_call(
    _scale_body, out_shape=jax.ShapeDtypeStruct((NPAD, D), jnp.float32))
_layer = pl.pallas_call(
    _layer_body, out_shape=jax.ShapeDtypeStruct((NPAD, H), jnp.float32))
_final = pl.pallas_call(
    _final_body, out_shape=jax.ShapeDtypeStruct((N, C), jnp.float32))


@jax.jit
def kernel(x, edge_index, W1, b1, g1, be1, W2, b2, g2, be2, Wc, bc):
    pad = (N + (jnp.arange(EP - E, dtype=jnp.int32) % NPADROWS))[None, :]
    e_pad = jnp.concatenate([edge_index, jnp.broadcast_to(pad, (2, EP - E))],
                            axis=1)
    e_deg = e_pad.reshape(2, NS, DNCH, ECH)
    e_agg = e_pad.reshape(2, NC, NS, ANCH, AECH)
    zeros1 = jnp.zeros((NPAD,), jnp.float32)
    zrows = jnp.zeros((RPT, D), jnp.float32)

    cnt = _deg_kernel(e_deg, zeros1)
    co = cnt[0, :N].reshape(N, 1)
    ci = cnt[1, :N].reshape(N, 1)

    xs = _scale(x, co)
    m1 = _agg_kernel(xs, e_agg, zrows)
    h1 = _layer(m1, ci, co, W1, b1.reshape(1, H), g1.reshape(1, H),
                be1.reshape(1, H))
    m2 = _agg_kernel(h1, e_agg, zrows)
    out = _final(m2, ci, W2, b2.reshape(1, H), g2.reshape(1, H),
                 be2.reshape(1, H), Wc, bc.reshape(1, C))
    return out
